# baseline (speedup 1.0000x reference)
import functools

import jax
import jax.numpy as jnp
from jax import lax
from jax.experimental import pallas as pl
from jax.experimental.pallas import tpu as pltpu, tpu_sc as plsc

_LANES = 16
_WROWS = 32


def _make_gather(B, C, num_workers, num_cores):
    b_per_w = B // num_workers
    n_win = b_per_w // _WROWS
    mesh = plsc.VectorSubcoreMesh(core_axis_name="c", subcore_axis_name="s")

    @functools.partial(
        pl.kernel,
        out_type=jax.ShapeDtypeStruct((B,), jnp.float32),
        mesh=mesh,
        scratch_types=[
            pltpu.VMEM((b_per_w,), jnp.int32),
            pltpu.VMEM((_WROWS, C), jnp.float32),
            pltpu.VMEM((_WROWS, C), jnp.float32),
            pltpu.VMEM((b_per_w,), jnp.float32),
            pltpu.SemaphoreType.DMA,
            pltpu.SemaphoreType.DMA,
        ],
        compiler_params=pltpu.CompilerParams(
            use_tc_tiling_on_sc=True, needs_layout_passes=False
        ),
    )
    def gather_kernel(x_hbm, y_hbm, out_hbm, y_v, buf0, buf1, vals_v, s0, s1):
        wid = lax.axis_index("s") * num_cores + lax.axis_index("c")
        base = wid * b_per_w
        pltpu.sync_copy(y_hbm.at[pl.ds(base, b_per_w)], y_v)
        bufs = (buf0, buf1)
        sems = (s0, s1)

        def select(w):
            buf = bufs[w % 2]
            for i in range(_WROWS // _LANES):
                off = w * _WROWS + i * _LANES
                rows = lax.iota(jnp.int32, _LANES) + i * _LANES
                cols = y_v[pl.ds(off, _LANES)]
                vals_v[pl.ds(off, _LANES)] = plsc.load_gather(buf, [rows, cols])

        descs = [None, None]
        for w in range(n_win):
            descs[w % 2] = pltpu.async_copy(
                x_hbm.at[pl.ds(base + w * _WROWS, _WROWS), :],
                bufs[w % 2],
                sems[w % 2],
            )
            if w >= 1:
                descs[(w - 1) % 2].wait()
                select(w - 1)
        descs[(n_win - 1) % 2].wait()
        select(n_win - 1)
        pltpu.sync_copy(vals_v, out_hbm.at[pl.ds(base, b_per_w)])

    return gather_kernel


def kernel(logits, y):
    B, C = logits.shape
    info = plsc.get_sparse_core_info()
    num_workers = info.num_cores * info.num_subcores
    y32 = y.astype(jnp.int32)
    return _make_gather(B, C, num_workers, info.num_cores)(logits, y32)


# speedup vs baseline: 1.0396x; 1.0396x over previous
import functools

import jax
import jax.numpy as jnp
from jax import lax
from jax.experimental import pallas as pl
from jax.experimental.pallas import tpu as pltpu, tpu_sc as plsc

_LANES = 16
_WROWS = 32
_BR = 2048
_LC = 128
_SPLIT = 10240


def _make_sc_gather(B, C, row0, num_workers, num_cores):
    rows = B - row0
    b_per_w = rows // num_workers
    n_win = b_per_w // _WROWS
    mesh = plsc.VectorSubcoreMesh(core_axis_name="c", subcore_axis_name="s")

    @functools.partial(
        pl.kernel,
        out_type=jax.ShapeDtypeStruct((rows,), jnp.float32),
        mesh=mesh,
        scratch_types=[
            pltpu.VMEM((b_per_w,), jnp.int32),
            pltpu.VMEM((_WROWS, C), jnp.float32),
            pltpu.VMEM((_WROWS, C), jnp.float32),
            pltpu.VMEM((b_per_w,), jnp.float32),
            pltpu.SemaphoreType.DMA,
            pltpu.SemaphoreType.DMA,
        ],
        compiler_params=pltpu.CompilerParams(
            use_tc_tiling_on_sc=True, needs_layout_passes=False
        ),
    )
    def gather_kernel(x_hbm, y_hbm, out_hbm, y_v, buf0, buf1, vals_v, s0, s1):
        wid = lax.axis_index("s") * num_cores + lax.axis_index("c")
        base = wid * b_per_w
        pltpu.sync_copy(y_hbm.at[pl.ds(row0 + base, b_per_w)], y_v)
        bufs = (buf0, buf1)
        sems = (s0, s1)

        def select(w):
            buf = bufs[w % 2]
            for i in range(_WROWS // _LANES):
                off = w * _WROWS + i * _LANES
                rows_v = lax.iota(jnp.int32, _LANES) + i * _LANES
                cols = y_v[pl.ds(off, _LANES)]
                vals_v[pl.ds(off, _LANES)] = plsc.load_gather(
                    buf, [rows_v, cols]
                )

        descs = [None, None]
        for w in range(n_win):
            descs[w % 2] = pltpu.async_copy(
                x_hbm.at[pl.ds(row0 + base + w * _WROWS, _WROWS), :],
                bufs[w % 2],
                sems[w % 2],
            )
            if w >= 1:
                descs[(w - 1) % 2].wait()
                select(w - 1)
        descs[(n_win - 1) % 2].wait()
        select(n_win - 1)
        pltpu.sync_copy(vals_v, out_hbm.at[pl.ds(base, b_per_w)])

    return gather_kernel


def _tc_select_kernel(y_ref, x_ref, o_ref):
    BR, C = x_ref.shape
    yb = y_ref[...].reshape(BR, 1)
    acc = jnp.zeros((BR, _LC), jnp.float32)
    for k in range(0, C, _LC):
        w = min(_LC, C - k)
        ids = jax.lax.broadcasted_iota(jnp.int32, (BR, w), 1) + k
        hit = jnp.where(ids == yb, x_ref[:, k:k + w], 0.0)
        if w < _LC:
            hit = jnp.pad(hit, ((0, 0), (0, _LC - w)))
        acc = acc + hit
    o_ref[...] = jnp.sum(acc, axis=1)


def _tc_select(logits, y32, n_rows):
    C = logits.shape[1]
    return pl.pallas_call(
        _tc_select_kernel,
        grid=(n_rows // _BR,),
        in_specs=[
            pl.BlockSpec((_BR,), lambda i: (i,)),
            pl.BlockSpec((_BR, C), lambda i: (i, 0)),
        ],
        out_specs=pl.BlockSpec((_BR,), lambda i: (i,)),
        out_shape=jax.ShapeDtypeStruct((n_rows,), jnp.float32),
    )(y32, logits)


def kernel(logits, y):
    B, C = logits.shape
    y32 = y.astype(jnp.int32)
    info = plsc.get_sparse_core_info()
    num_workers = info.num_cores * info.num_subcores
    sc_part = _make_sc_gather(B, C, _SPLIT, num_workers, info.num_cores)(
        logits, y32
    )
    tc_part = _tc_select(logits, y32, _SPLIT)
    return jnp.concatenate([tc_part, sc_part])


# speedup vs baseline: 1.2258x; 1.1791x over previous
import functools

import jax
import jax.numpy as jnp
from jax.experimental import pallas as pl
from jax.experimental.pallas import tpu as pltpu

_CR = 256
_NBUF = 8
_LC = 128


def _select_chunk(yb, x):
    BR, C = x.shape
    acc = jnp.zeros((BR, _LC), jnp.float32)
    for k in range(0, C, _LC):
        w = min(_LC, C - k)
        ids = jax.lax.broadcasted_iota(jnp.int32, (BR, w), 1) + k
        hit = jnp.where(ids == yb, x[:, k:k + w], 0.0)
        if w < _LC:
            hit = jnp.pad(hit, ((0, 0), (0, _LC - w)))
        acc = acc + hit
    return jnp.sum(acc, axis=1)


def _ring_kernel(y_ref, x_hbm, o_ref, bufs, sems):
    B, C = x_hbm.shape
    n_chunks = B // _CR

    def start(i):
        pltpu.make_async_copy(
            x_hbm.at[pl.ds(i * _CR, _CR), :],
            bufs.at[i % _NBUF],
            sems.at[i % _NBUF],
        ).start()

    for b in range(min(_NBUF, n_chunks)):
        start(b)
    for i in range(n_chunks):
        pltpu.make_async_copy(
            x_hbm.at[pl.ds(i * _CR, _CR), :],
            bufs.at[i % _NBUF],
            sems.at[i % _NBUF],
        ).wait()
        yb = y_ref[pl.ds(i * _CR, _CR)].reshape(_CR, 1)
        o_ref[pl.ds(i * _CR, _CR)] = _select_chunk(yb, bufs[i % _NBUF])
        if i + _NBUF < n_chunks:
            start(i + _NBUF)


def kernel(logits, y):
    B, C = logits.shape
    y32 = y.astype(jnp.int32)
    return pl.pallas_call(
        _ring_kernel,
        in_specs=[
            pl.BlockSpec(memory_space=pltpu.VMEM),
            pl.BlockSpec(memory_space=pl.ANY),
        ],
        out_specs=pl.BlockSpec(memory_space=pltpu.VMEM),
        out_shape=jax.ShapeDtypeStruct((B,), jnp.float32),
        scratch_shapes=[
            pltpu.VMEM((_NBUF, _CR, C), jnp.float32),
            pltpu.SemaphoreType.DMA((_NBUF,)),
        ],
    )(y32, logits)
